# Initial kernel scaffold; baseline (speedup 1.0000x reference)
import jax
import jax.numpy as jnp
from jax.experimental import pallas as pl


def kernel(x, lengths, uni, table1, table2, table3, mask1, mask2, mask3):
    raise NotImplementedError("write your pallas kernel here")


# speedup vs baseline: 1.6762x; 1.6762x over previous
import functools

import jax
import jax.numpy as jnp
from jax import lax
from jax.experimental import pallas as pl
from jax.experimental.pallas import tpu as pltpu
from jax.experimental.pallas import tpu_sc as plsc

V = 60
VP = 64
NUM_WORKERS = 32
LANES = 16
CHUNK = 128
NSUB = 5


def _make_kernel(B, T):
    rows_w = B // NUM_WORKERS
    n_chunks = rows_w // CHUNK
    nsub = rows_w * NSUB
    nsub_chunks = nsub // CHUNK
    groups = rows_w // LANES
    mesh = plsc.VectorSubcoreMesh(core_axis_name="c", subcore_axis_name="s")

    @functools.partial(
        pl.kernel,
        out_type=jax.ShapeDtypeStruct((B, VP), jnp.float32),
        mesh=mesh,
        compiler_params=pltpu.CompilerParams(
            needs_layout_passes=False, use_tc_tiling_on_sc=False),
        scratch_types=[
            pltpu.VMEM((rows_w * T,), jnp.int32),
            pltpu.VMEM((rows_w,), jnp.int32),
            pltpu.VMEM((V + 1, VP), jnp.float32),
            pltpu.VMEM((V,), jnp.int32),
            pltpu.VMEM((V * V,), jnp.int32),
            pltpu.VMEM((n_chunks, CHUNK), jnp.int32),
            pltpu.VMEM((n_chunks, CHUNK), jnp.int32),
            pltpu.VMEM((nsub,), jnp.int32),
            pltpu.VMEM((n_chunks, CHUNK), jnp.int32),
            pltpu.VMEM((rows_w,), jnp.int32),
            pltpu.VMEM((rows_w,), jnp.int32),
            pltpu.VMEM((rows_w, VP), jnp.float32),
            pltpu.VMEM((nsub, LANES), jnp.float32),
            pltpu.SemaphoreType.DMA,
            pltpu.SemaphoreType.DMA,
            pltpu.SemaphoreType.DMA,
        ],
    )
    def ngram_kernel(x_hbm, len_hbm, t01_hbm, m1_hbm, m2_hbm, t2_hbm, t3w_hbm,
                     m3_hbm, out_hbm, xv, lenv, t01v, m1v, m2v, k2v, k3v,
                     ksv, m3v, codev, sv, r2v, r3w, sem3, sem2, semm3):
        cid = lax.axis_index("c")
        sid = lax.axis_index("s")
        wid = cid * 16 + sid
        base = wid * rows_w

        pltpu.sync_copy(x_hbm.at[pl.ds(base * T, rows_w * T)], xv)
        pltpu.sync_copy(len_hbm.at[pl.ds(base, rows_w)], lenv)
        pltpu.sync_copy(t01_hbm, t01v)
        pltpu.sync_copy(m1_hbm, m1v)
        pltpu.sync_copy(m2_hbm, m2v)

        for g in range(groups):
            lens = lenv[pl.ds(g * LANES, LANES)]
            rows = lax.broadcasted_iota(jnp.int32, (LANES,), 0) + g * LANES
            p1 = jnp.clip(lens - 1, 0, T - 1)
            p2 = jnp.clip(lens - 2, 0, T - 1)
            p3 = jnp.clip(lens - 3, 0, T - 1)
            rbase = rows * T
            a = plsc.load_gather(xv, [rbase + p1])
            b = plsc.load_gather(xv, [rbase + p2])
            c = plsc.load_gather(xv, [rbase + p3])
            key2 = b * V + a
            key3 = c * (V * V) + key2
            m1bit = plsc.load_gather(m1v, [a])
            m2bit = plsc.load_gather(m2v, [key2])
            take2 = jnp.logical_and(lens >= 2, m2bit != 0)
            take1 = jnp.logical_and(lens >= 1, m1bit != 0)
            src01 = jnp.where(take1, a + 1, 0)
            ch = g * LANES // CHUNK
            off = (g * LANES) % CHUNK
            k2v[ch, pl.ds(off, LANES)] = key2
            k3v[ch, pl.ds(off, LANES)] = key3
            q0 = (key3 * 15) >> 2
            sv[pl.ds(g * LANES, LANES)] = key3 * 60 - q0 * 16
            pbase = rows * NSUB
            nwin = V * V * V * V // LANES
            for j in range(NSUB):
                plsc.store_scatter(ksv, [pbase + j],
                                   jnp.minimum(q0 + j, nwin - 1))
            codev[pl.ds(g * LANES, LANES)] = jnp.where(take2, 1, 2 + src01)

        copies = []
        for ch in range(nsub_chunks):
            copies.append(pltpu.async_copy(
                t3w_hbm.at[ksv.at[pl.ds(ch * CHUNK, CHUNK)]],
                r3w.at[pl.ds(ch * CHUNK, CHUNK)], sem3))
        for ch in range(n_chunks):
            row0 = ch * CHUNK
            copies.append(pltpu.async_copy(
                t2_hbm.at[k2v.at[ch]], r2v.at[pl.ds(row0, CHUNK)], sem2))
            copies.append(pltpu.async_copy(
                m3_hbm.at[k3v.at[ch]], m3v.at[ch], semm3))
        for cp in copies:
            cp.wait()

        for g in range(groups):
            ch = g * LANES // CHUNK
            off = (g * LANES) % CHUNK
            lens = lenv[pl.ds(g * LANES, LANES)]
            m3bit = m3v[ch, pl.ds(off, LANES)]
            take3 = jnp.logical_and(lens >= 3, m3bit != 0)
            old = codev[pl.ds(g * LANES, LANES)]
            codev[pl.ds(g * LANES, LANES)] = jnp.where(take3, 0, old)

        lane = lax.broadcasted_iota(jnp.int32, (LANES,), 0)

        def g_body(g, carry):
            row0 = g * LANES
            codes = codev[pl.ds(row0, LANES)]
            svec = sv[pl.ds(row0, LANES)]
            for l in range(LANES):
                c = codes[l]
                r = row0 + l

                @pl.when(c == 0)
                def _():
                    pos0 = r * (NSUB * LANES) + svec[l] + lane
                    for o in (0, 16, 32, V - LANES):
                        pos = pos0 + o
                        r2v[r, pl.ds(o, LANES)] = plsc.load_gather(
                            r3w, [pos >> 4, pos & 15])

                @pl.when(c >= 2)
                def _():
                    src = c - 2
                    for o in (0, 16, 32, 48):
                        r2v[r, pl.ds(o, LANES)] = t01v[src, pl.ds(o, LANES)]

            return carry

        lax.fori_loop(0, groups, g_body, 0)

        pltpu.sync_copy(r2v, out_hbm.at[pl.ds(base, rows_w)])

    return ngram_kernel


@jax.jit
def kernel(x, lengths, uni, table1, table2, table3, mask1, mask2, mask3):
    B, T = x.shape
    t01 = jnp.pad(jnp.concatenate([uni[None, :], table1], axis=0),
                  ((0, 0), (0, VP - V)))
    t2p = jnp.pad(table2, ((0, 0), (0, VP - V)))
    t3w = table3.reshape(-1, LANES)
    m1 = mask1.astype(jnp.int32)
    m2 = mask2.astype(jnp.int32)
    m3 = mask3.astype(jnp.int32)
    out = _make_kernel(B, T)(
        x.reshape(-1), lengths.astype(jnp.int32), t01, m1, m2, t2p, t3w, m3)
    return out[:, None, :V]
